# baseline (speedup 1.0000x reference)
import functools
import math

import jax
import jax.numpy as jnp
from jax.experimental import pallas as pl
from jax.experimental.pallas import tpu as pltpu
from jax.experimental.pallas import tpu_sc as plsc

BOX = 27.27
INV_BOX = 1.0 / BOX
POS_MEAN = 13.635
POS_VAR = 61.97
POS_STD = math.sqrt(POS_VAR)
K_NEIGH = 16
N_LAYERS = 4
HID = 128
N_NODES = 10000

C_PAD = 10112
R_KNN = 200
B_NODE = 400
E_EDGES = N_NODES * K_NEIGH
E_PAD = 163840


def _knn_body(pv_ref, pvT_ref, nbr_ref, d2_ref):
    d2 = jnp.zeros((R_KNN, C_PAD), jnp.float32)
    for d in range(3):
        qd = pv_ref[:, d:d + 1] * POS_STD + POS_MEAN
        cd = pvT_ref[d:d + 1, :] * POS_STD + POS_MEAN
        diff = qd - cd
        w = diff - BOX * jnp.round(diff * INV_BOX)
        d2 = d2 + w * w
    lane = jax.lax.broadcasted_iota(jnp.int32, (1, C_PAD), 1)
    d2 = jnp.where(lane < N_NODES, d2, jnp.inf)
    row0 = pl.program_id(0) * R_KNN
    rows = row0 + jax.lax.broadcasted_iota(jnp.int32, (R_KNN, 1), 0)
    d2_ref[...] = jnp.where(lane == rows, jnp.inf, d2)
    out_lane = jax.lax.broadcasted_iota(jnp.int32, (1, 128), 1)

    def step(k, acc):
        d2 = d2_ref[...]
        idx = jnp.argmin(d2, axis=1)[:, None]
        acc = jnp.where(out_lane == k, idx, acc)
        d2_ref[...] = jnp.where(lane == idx, jnp.inf, d2)
        return acc

    acc = jax.lax.fori_loop(0, K_NEIGH, step,
                            jnp.zeros((R_KNN, 128), jnp.int32))
    nbr_ref[...] = acc


def _knn_call(pv8, pvT, interpret=False):
    grid = N_NODES // R_KNN
    return pl.pallas_call(
        _knn_body,
        grid=(grid,),
        in_specs=[
            pl.BlockSpec((R_KNN, 8), lambda i: (i, 0)),
            pl.BlockSpec((8, C_PAD), lambda i: (0, 0)),
        ],
        out_specs=pl.BlockSpec((R_KNN, 128), lambda i: (i, 0)),
        out_shape=jax.ShapeDtypeStruct((N_NODES, 128), jnp.int32),
        scratch_shapes=[pltpu.VMEM((R_KNN, C_PAD), jnp.float32)],
        interpret=interpret,
    )(pv8, pvT)


def _ln(x):
    mu = jnp.mean(x, axis=-1, keepdims=True)
    var = jnp.mean((x - mu) * (x - mu), axis=-1, keepdims=True)
    return (x - mu) / jnp.sqrt(var + 1e-5)


def _embed_body(pv_ref, We1_ref, be1_ref, We2_ref, be2_ref, Wh_ref, bh_ref,
                h_ref, hw_ref, pos_ref):
    pv = pv_ref[...]
    t = jnp.zeros((pv.shape[0], HID), jnp.float32)
    for j in range(6):
        t = t + pv[:, j:j + 1] * We1_ref[j:j + 1, :]
    h1 = jnp.maximum(t + be1_ref[...], 0.0)
    h = jnp.dot(h1, We2_ref[...], preferred_element_type=jnp.float32)
    h = _ln(h + be2_ref[...])
    h_ref[...] = h
    hw_ref[...] = jnp.dot(h, Wh_ref[...],
                          preferred_element_type=jnp.float32) + bh_ref[...]
    lane = jax.lax.broadcasted_iota(jnp.int32, (1, HID), 1)
    posb = jnp.zeros((pv.shape[0], HID), jnp.float32)
    for j in range(3):
        posb = posb + jnp.where(lane == j,
                                pv[:, j:j + 1] * POS_STD + POS_MEAN, 0.0)
    pos_ref[...] = posb


def _embed_call(pv8, We1p, be1, We2, be2, Wh, bh, interpret=False):
    grid = N_NODES // B_NODE
    w128 = pl.BlockSpec((HID, HID), lambda i: (0, 0))
    b128 = pl.BlockSpec((1, HID), lambda i: (0, 0))
    return pl.pallas_call(
        _embed_body,
        grid=(grid,),
        in_specs=[
            pl.BlockSpec((B_NODE, 8), lambda i: (i, 0)),
            pl.BlockSpec((8, HID), lambda i: (0, 0)),
            b128, w128, b128, w128, b128,
        ],
        out_specs=[
            pl.BlockSpec((B_NODE, HID), lambda i: (i, 0)),
            pl.BlockSpec((B_NODE, HID), lambda i: (i, 0)),
            pl.BlockSpec((B_NODE, HID), lambda i: (i, 0)),
        ],
        out_shape=[
            jax.ShapeDtypeStruct((N_NODES, HID), jnp.float32),
            jax.ShapeDtypeStruct((N_NODES, HID), jnp.float32),
            jax.ShapeDtypeStruct((N_NODES, HID), jnp.float32),
        ],
        interpret=interpret,
    )(pv8, We1p, be1, We2, be2, Wh, bh)


N_WORKERS = 32
CHUNK = 128


def _sc_gather(table, idx_flat):
    ep, d = idx_flat.shape[0], table.shape[1]
    DEPTH = 4 if d <= 128 else 2
    mesh = plsc.VectorSubcoreMesh(core_axis_name="c", subcore_axis_name="s",
                                  num_cores=1)
    rows_w = ep // 16
    n_grp = rows_w // (CHUNK * DEPTH)

    @functools.partial(
        pl.kernel,
        out_type=jax.ShapeDtypeStruct((ep, d), jnp.float32),
        mesh=mesh,
        scratch_types=(
            [pltpu.VMEM((rows_w,), jnp.int32)]
            + [pltpu.VMEM((CHUNK, d), jnp.float32) for _ in range(DEPTH)]
            + [pltpu.SemaphoreType.DMA for _ in range(2 * DEPTH)]
        ),
    )
    def k(table_hbm, idx_hbm, out_hbm, idx_v, *rest):
        bufs = rest[:DEPTH]
        gsem = rest[DEPTH:2 * DEPTH]
        osem = rest[2 * DEPTH:]
        wid = jax.lax.axis_index("s")
        base = wid * rows_w
        pltpu.sync_copy(idx_hbm.at[pl.ds(base, rows_w)], idx_v)

        @pl.loop(0, n_grp)
        def _(i):
            off = i * (CHUNK * DEPTH)
            gh = []
            for b in range(DEPTH):
                gh.append(pltpu.async_copy(
                    table_hbm.at[idx_v.at[pl.ds(off + b * CHUNK, CHUNK)]],
                    bufs[b], gsem[b]))
            oh = []
            for b in range(DEPTH):
                gh[b].wait()
                oh.append(pltpu.async_copy(
                    bufs[b], out_hbm.at[pl.ds(base + off + b * CHUNK, CHUNK)],
                    osem[b]))
            for b in range(DEPTH):
                oh[b].wait()

    return k(table, idx_flat)


def _edge_body(ps_ref, pd_ref, Wq1_ref, bq1_ref, Wq2_ref, bq2_ref, e_ref):
    b = pd_ref.shape[0]
    ps = ps_ref[...]
    pd = pd_ref[...]
    pdr = jnp.broadcast_to(pd[:, None, :], (b, K_NEIGH, HID))
    pdr = pdr.reshape(b * K_NEIGH, HID)
    d = ps - pdr
    rel = d - BOX * jnp.round(d * INV_BOX)
    lane16 = jax.lax.broadcasted_iota(jnp.int32, (1, HID), 1)
    rel = jnp.where(lane16 < 3, rel, 0.0)
    rn = jnp.sqrt(jnp.sum(rel * rel, axis=1, keepdims=True) + 1e-12)
    t = (rel[:, 0:1] * Wq1_ref[0:1, :] + rel[:, 1:2] * Wq1_ref[1:2, :]
         + rel[:, 2:3] * Wq1_ref[2:3, :] + rn * Wq1_ref[3:4, :])
    h1 = jnp.maximum(t + bq1_ref[...], 0.0)
    e_ref[...] = jnp.dot(h1, Wq2_ref[...],
                         preferred_element_type=jnp.float32) + bq2_ref[...]


def _edge_call(possrc, pos16, Wq1, bq1, Wq2, bq2, interpret=False):
    grid = N_NODES // B_NODE
    w128 = pl.BlockSpec((HID, HID), lambda i: (0, 0))
    b128 = pl.BlockSpec((1, HID), lambda i: (0, 0))
    return pl.pallas_call(
        _edge_body,
        grid=(grid,),
        in_specs=[
            pl.BlockSpec((B_NODE * K_NEIGH, HID), lambda i: (i, 0)),
            pl.BlockSpec((B_NODE, HID), lambda i: (i, 0)),
            pl.BlockSpec((4, HID), lambda i: (0, 0)),
            b128, w128, b128,
        ],
        out_specs=pl.BlockSpec((B_NODE * K_NEIGH, HID), lambda i: (i, 0)),
        out_shape=jax.ShapeDtypeStruct((E_EDGES, HID), jnp.float32),
        interpret=interpret,
    )(possrc, pos16, Wq1, bq1, Wq2, bq2)


def _layer_body(g_ref, e_ref, h_ref, Wme_ref, Wm2_ref, bm2_ref, Wu_ref, bu_ref,
                Wn_ref, bn_ref, Wn2_ref, bn2_ref, h_out_ref, hw_out_ref,
                *, last):
    b = h_ref.shape[0]
    eW = jnp.dot(e_ref[...], Wme_ref[...], preferred_element_type=jnp.float32)
    m1 = jnp.maximum(g_ref[...] + eW, 0.0)
    m = jnp.dot(m1, Wm2_ref[...],
                preferred_element_type=jnp.float32) + bm2_ref[...]
    agg = jnp.sum(m.reshape(b, K_NEIGH, HID), axis=1)
    u = jnp.maximum(jnp.dot(agg, Wu_ref[...],
                            preferred_element_type=jnp.float32) + bu_ref[...],
                    0.0)
    hn = _ln(h_ref[...] + u)
    h_out_ref[...] = hn
    if last:
        d1 = jnp.maximum(jnp.dot(hn, Wn_ref[...],
                                 preferred_element_type=jnp.float32)
                         + bn_ref[...], 0.0)
        hw_out_ref[...] = jnp.dot(d1, Wn2_ref[...],
                                  preferred_element_type=jnp.float32) \
            + bn2_ref[...]
    else:
        hw_out_ref[...] = jnp.dot(hn, Wn_ref[...],
                                  preferred_element_type=jnp.float32) \
            + bn_ref[...]


def _layer_call(g, g_col, e, h, Wme, Wm2, bm2, Wu, bu, Wn, bn, Wn2, bn2, last,
                interpret=False):
    grid = N_NODES // B_NODE
    w128 = pl.BlockSpec((HID, HID), lambda i: (0, 0))
    b128 = pl.BlockSpec((1, HID), lambda i: (0, 0))
    return pl.pallas_call(
        functools.partial(_layer_body, last=last),
        grid=(grid,),
        in_specs=[
            pl.BlockSpec((B_NODE * K_NEIGH, HID),
                         lambda i, c=g_col: (i, c)),
            pl.BlockSpec((B_NODE * K_NEIGH, HID), lambda i: (i, 0)),
            pl.BlockSpec((B_NODE, HID), lambda i: (i, 0)),
            w128, w128, b128, w128, b128, w128, b128, w128, b128,
        ],
        out_specs=[
            pl.BlockSpec((B_NODE, HID), lambda i: (i, 0)),
            pl.BlockSpec((B_NODE, HID), lambda i: (i, 0)),
        ],
        out_shape=[
            jax.ShapeDtypeStruct((N_NODES, HID), jnp.float32),
            jax.ShapeDtypeStruct((N_NODES, HID), jnp.float32),
        ],
        interpret=interpret,
    )(g, e, h, Wme, Wm2, bm2, Wu, bu, Wn, bn, Wn2, bn2)


def kernel(pos_vel, We1, be1, We2, be2, Wq1, bq1, Wq2, bq2, Wm1, bm1, Wm2,
           bm2, Wu, bu, Wd1, bd1, Wd2, bd2, t):
    f32 = jnp.float32
    pv8 = jnp.pad(pos_vel, ((0, 0), (0, 2)))
    pvT = jnp.pad(pos_vel.T, ((0, 2), (0, C_PAD - N_NODES)))
    We1p = jnp.pad(We1, ((0, 2), (0, 0)))

    nbr = _knn_call(pv8, pvT)
    h0, hw0, pos16 = _embed_call(
        pv8, We1p, be1.reshape(1, HID), We2, be2.reshape(1, HID),
        Wm1[0, :HID, :], bm1[0].reshape(1, HID))

    src = nbr[:, :K_NEIGH].reshape(-1)
    src_pad = jnp.pad(src, (0, E_PAD - E_EDGES))

    posw0 = jnp.concatenate([pos16, hw0], axis=1)
    gath0 = _sc_gather(posw0, src_pad)
    e = _edge_call(gath0, pos16, Wq1, bq1.reshape(1, HID), Wq2,
                   bq2.reshape(1, HID))

    Wd2p = jnp.pad(Wd2, ((0, 0), (0, HID - 3)))
    bd2p = jnp.pad(bd2, (0, HID - 3)).reshape(1, HID)

    h, hw = h0, hw0
    for l in range(N_LAYERS):
        if l == 0:
            g, g_col = gath0, 1
        else:
            g, g_col = _sc_gather(hw, src_pad), 0
        last = l == N_LAYERS - 1
        if last:
            Wn, bn = Wd1, bd1.reshape(1, HID)
            Wn2, bn2 = Wd2p, bd2p
        else:
            Wn, bn = Wm1[l + 1, :HID, :], bm1[l + 1].reshape(1, HID)
            Wn2, bn2 = Wd2p, bd2p
        h, hw = _layer_call(
            g, g_col, e, h, Wm1[l, HID:, :], Wm2[l], bm2[l].reshape(1, HID),
            Wu[l], bu[l].reshape(1, HID), Wn, bn, Wn2, bn2, last)

    return hw[:, :3].astype(f32)


# speedup vs baseline: 1.0441x; 1.0441x over previous
import functools
import math

import jax
import jax.numpy as jnp
from jax.experimental import pallas as pl
from jax.experimental.pallas import tpu as pltpu
from jax.experimental.pallas import tpu_sc as plsc

BOX = 27.27
INV_BOX = 1.0 / BOX
POS_MEAN = 13.635
POS_VAR = 61.97
POS_STD = math.sqrt(POS_VAR)
K_NEIGH = 16
N_LAYERS = 4
HID = 128
N_NODES = 10000

C_PAD = 10112
R_KNN = 200
B_NODE = 400
E_EDGES = N_NODES * K_NEIGH
E_PAD = 163840


def _knn_body(pv_ref, pvT_ref, nbr_ref, d2_ref):
    d2 = jnp.zeros((R_KNN, C_PAD), jnp.float32)
    for d in range(3):
        qd = pv_ref[:, d:d + 1] * POS_STD + POS_MEAN
        cd = pvT_ref[d:d + 1, :] * POS_STD + POS_MEAN
        diff = qd - cd
        w = diff - BOX * jnp.round(diff * INV_BOX)
        d2 = d2 + w * w
    lane = jax.lax.broadcasted_iota(jnp.int32, (1, C_PAD), 1)
    d2 = jnp.where(lane < N_NODES, d2, jnp.inf)
    row0 = pl.program_id(0) * R_KNN
    rows = row0 + jax.lax.broadcasted_iota(jnp.int32, (R_KNN, 1), 0)
    d2_ref[...] = jnp.where(lane == rows, jnp.inf, d2)
    out_lane = jax.lax.broadcasted_iota(jnp.int32, (1, 128), 1)

    def step(k, acc):
        d2 = d2_ref[...]
        idx = jnp.argmin(d2, axis=1)[:, None]
        acc = jnp.where(out_lane == k, idx, acc)
        d2_ref[...] = jnp.where(lane == idx, jnp.inf, d2)
        return acc

    acc = jax.lax.fori_loop(0, K_NEIGH, step,
                            jnp.zeros((R_KNN, 128), jnp.int32))
    nbr_ref[...] = acc


def _knn_call(pv8, pvT, interpret=False):
    grid = N_NODES // R_KNN
    return pl.pallas_call(
        _knn_body,
        grid=(grid,),
        in_specs=[
            pl.BlockSpec((R_KNN, 8), lambda i: (i, 0)),
            pl.BlockSpec((8, C_PAD), lambda i: (0, 0)),
        ],
        out_specs=pl.BlockSpec((R_KNN, 128), lambda i: (i, 0)),
        out_shape=jax.ShapeDtypeStruct((N_NODES, 128), jnp.int32),
        scratch_shapes=[pltpu.VMEM((R_KNN, C_PAD), jnp.float32)],
        interpret=interpret,
    )(pv8, pvT)


def _ln(x):
    mu = jnp.mean(x, axis=-1, keepdims=True)
    var = jnp.mean((x - mu) * (x - mu), axis=-1, keepdims=True)
    return (x - mu) / jnp.sqrt(var + 1e-5)


def _embed_body(pv_ref, We1_ref, be1_ref, We2_ref, be2_ref, Wh_ref, bh_ref,
                h_ref, hw_ref, pos_ref):
    pv = pv_ref[...]
    t = jnp.zeros((pv.shape[0], HID), jnp.float32)
    for j in range(6):
        t = t + pv[:, j:j + 1] * We1_ref[j:j + 1, :]
    h1 = jnp.maximum(t + be1_ref[...], 0.0)
    h = jnp.dot(h1, We2_ref[...], preferred_element_type=jnp.float32)
    h = _ln(h + be2_ref[...])
    h_ref[...] = h
    hw_ref[...] = jnp.dot(h, Wh_ref[...],
                          preferred_element_type=jnp.float32) + bh_ref[...]
    lane = jax.lax.broadcasted_iota(jnp.int32, (1, HID), 1)
    posb = jnp.zeros((pv.shape[0], HID), jnp.float32)
    for j in range(3):
        posb = posb + jnp.where(lane == j,
                                pv[:, j:j + 1] * POS_STD + POS_MEAN, 0.0)
    pos_ref[...] = posb


def _embed_call(pv8, We1p, be1, We2, be2, Wh, bh, interpret=False):
    grid = N_NODES // B_NODE
    w128 = pl.BlockSpec((HID, HID), lambda i: (0, 0))
    b128 = pl.BlockSpec((1, HID), lambda i: (0, 0))
    return pl.pallas_call(
        _embed_body,
        grid=(grid,),
        in_specs=[
            pl.BlockSpec((B_NODE, 8), lambda i: (i, 0)),
            pl.BlockSpec((8, HID), lambda i: (0, 0)),
            b128, w128, b128, w128, b128,
        ],
        out_specs=[
            pl.BlockSpec((B_NODE, HID), lambda i: (i, 0)),
            pl.BlockSpec((B_NODE, HID), lambda i: (i, 0)),
            pl.BlockSpec((B_NODE, HID), lambda i: (i, 0)),
        ],
        out_shape=[
            jax.ShapeDtypeStruct((N_NODES, HID), jnp.float32),
            jax.ShapeDtypeStruct((N_NODES, HID), jnp.float32),
            jax.ShapeDtypeStruct((N_NODES, HID), jnp.float32),
        ],
        interpret=interpret,
    )(pv8, We1p, be1, We2, be2, Wh, bh)


N_WORKERS = 32
CHUNK = 128


def _sc_gather(table, idx_flat):
    ep, d = idx_flat.shape[0], table.shape[1]
    DEPTH = 4 if d <= 128 else 2
    mesh = plsc.VectorSubcoreMesh(core_axis_name="c", subcore_axis_name="s")
    rows_w = ep // N_WORKERS
    n_grp = rows_w // (CHUNK * DEPTH)

    @functools.partial(
        pl.kernel,
        out_type=jax.ShapeDtypeStruct((ep, d), jnp.float32),
        mesh=mesh,
        scratch_types=(
            [pltpu.VMEM((rows_w,), jnp.int32)]
            + [pltpu.VMEM((CHUNK, d), jnp.float32) for _ in range(DEPTH)]
            + [pltpu.SemaphoreType.DMA for _ in range(2 * DEPTH)]
        ),
    )
    def k(table_hbm, idx_hbm, out_hbm, idx_v, *rest):
        bufs = rest[:DEPTH]
        gsem = rest[DEPTH:2 * DEPTH]
        osem = rest[2 * DEPTH:]
        c = jax.lax.axis_index("c")
        s = jax.lax.axis_index("s")
        wid = s * 2 + c
        base = wid * rows_w
        pltpu.sync_copy(idx_hbm.at[pl.ds(base, rows_w)], idx_v)

        @pl.loop(0, n_grp)
        def _(i):
            off = i * (CHUNK * DEPTH)
            gh = []
            for b in range(DEPTH):
                gh.append(pltpu.async_copy(
                    table_hbm.at[idx_v.at[pl.ds(off + b * CHUNK, CHUNK)]],
                    bufs[b], gsem[b]))
            oh = []
            for b in range(DEPTH):
                gh[b].wait()
                oh.append(pltpu.async_copy(
                    bufs[b], out_hbm.at[pl.ds(base + off + b * CHUNK, CHUNK)],
                    osem[b]))
            for b in range(DEPTH):
                oh[b].wait()

    return k(table, idx_flat)


def _edge_body(ps_ref, pd_ref, Wq1_ref, bq1_ref, Wq2_ref, bq2_ref, e_ref):
    b = pd_ref.shape[0]
    ps = ps_ref[...]
    pd = pd_ref[...]
    pdr = jnp.broadcast_to(pd[:, None, :], (b, K_NEIGH, HID))
    pdr = pdr.reshape(b * K_NEIGH, HID)
    d = ps - pdr
    rel = d - BOX * jnp.round(d * INV_BOX)
    lane16 = jax.lax.broadcasted_iota(jnp.int32, (1, HID), 1)
    rel = jnp.where(lane16 < 3, rel, 0.0)
    rn = jnp.sqrt(jnp.sum(rel * rel, axis=1, keepdims=True) + 1e-12)
    t = (rel[:, 0:1] * Wq1_ref[0:1, :] + rel[:, 1:2] * Wq1_ref[1:2, :]
         + rel[:, 2:3] * Wq1_ref[2:3, :] + rn * Wq1_ref[3:4, :])
    h1 = jnp.maximum(t + bq1_ref[...], 0.0)
    e_ref[...] = jnp.dot(h1, Wq2_ref[...],
                         preferred_element_type=jnp.float32) + bq2_ref[...]


def _edge_call(possrc, pos16, Wq1, bq1, Wq2, bq2, interpret=False):
    grid = N_NODES // B_NODE
    w128 = pl.BlockSpec((HID, HID), lambda i: (0, 0))
    b128 = pl.BlockSpec((1, HID), lambda i: (0, 0))
    return pl.pallas_call(
        _edge_body,
        grid=(grid,),
        in_specs=[
            pl.BlockSpec((B_NODE * K_NEIGH, HID), lambda i: (i, 0)),
            pl.BlockSpec((B_NODE, HID), lambda i: (i, 0)),
            pl.BlockSpec((4, HID), lambda i: (0, 0)),
            b128, w128, b128,
        ],
        out_specs=pl.BlockSpec((B_NODE * K_NEIGH, HID), lambda i: (i, 0)),
        out_shape=jax.ShapeDtypeStruct((E_EDGES, HID), jnp.float32),
        interpret=interpret,
    )(possrc, pos16, Wq1, bq1, Wq2, bq2)


def _layer_body(g_ref, e_ref, h_ref, Wme_ref, Wm2_ref, bm2_ref, Wu_ref, bu_ref,
                Wn_ref, bn_ref, Wn2_ref, bn2_ref, h_out_ref, hw_out_ref,
                *, last):
    b = h_ref.shape[0]
    eW = jnp.dot(e_ref[...], Wme_ref[...], preferred_element_type=jnp.float32)
    m1 = jnp.maximum(g_ref[...] + eW, 0.0)
    m = jnp.dot(m1, Wm2_ref[...],
                preferred_element_type=jnp.float32) + bm2_ref[...]
    agg = jnp.sum(m.reshape(b, K_NEIGH, HID), axis=1)
    u = jnp.maximum(jnp.dot(agg, Wu_ref[...],
                            preferred_element_type=jnp.float32) + bu_ref[...],
                    0.0)
    hn = _ln(h_ref[...] + u)
    h_out_ref[...] = hn
    if last:
        d1 = jnp.maximum(jnp.dot(hn, Wn_ref[...],
                                 preferred_element_type=jnp.float32)
                         + bn_ref[...], 0.0)
        hw_out_ref[...] = jnp.dot(d1, Wn2_ref[...],
                                  preferred_element_type=jnp.float32) \
            + bn2_ref[...]
    else:
        hw_out_ref[...] = jnp.dot(hn, Wn_ref[...],
                                  preferred_element_type=jnp.float32) \
            + bn_ref[...]


def _layer_call(g, g_col, e, h, Wme, Wm2, bm2, Wu, bu, Wn, bn, Wn2, bn2, last,
                interpret=False):
    grid = N_NODES // B_NODE
    w128 = pl.BlockSpec((HID, HID), lambda i: (0, 0))
    b128 = pl.BlockSpec((1, HID), lambda i: (0, 0))
    return pl.pallas_call(
        functools.partial(_layer_body, last=last),
        grid=(grid,),
        in_specs=[
            pl.BlockSpec((B_NODE * K_NEIGH, HID),
                         lambda i, c=g_col: (i, c)),
            pl.BlockSpec((B_NODE * K_NEIGH, HID), lambda i: (i, 0)),
            pl.BlockSpec((B_NODE, HID), lambda i: (i, 0)),
            w128, w128, b128, w128, b128, w128, b128, w128, b128,
        ],
        out_specs=[
            pl.BlockSpec((B_NODE, HID), lambda i: (i, 0)),
            pl.BlockSpec((B_NODE, HID), lambda i: (i, 0)),
        ],
        out_shape=[
            jax.ShapeDtypeStruct((N_NODES, HID), jnp.float32),
            jax.ShapeDtypeStruct((N_NODES, HID), jnp.float32),
        ],
        interpret=interpret,
    )(g, e, h, Wme, Wm2, bm2, Wu, bu, Wn, bn, Wn2, bn2)


def kernel(pos_vel, We1, be1, We2, be2, Wq1, bq1, Wq2, bq2, Wm1, bm1, Wm2,
           bm2, Wu, bu, Wd1, bd1, Wd2, bd2, t):
    f32 = jnp.float32
    pv8 = jnp.pad(pos_vel, ((0, 0), (0, 2)))
    pvT = jnp.pad(pos_vel.T, ((0, 2), (0, C_PAD - N_NODES)))
    We1p = jnp.pad(We1, ((0, 2), (0, 0)))

    nbr = _knn_call(pv8, pvT)
    h0, hw0, pos16 = _embed_call(
        pv8, We1p, be1.reshape(1, HID), We2, be2.reshape(1, HID),
        Wm1[0, :HID, :], bm1[0].reshape(1, HID))

    src = nbr[:, :K_NEIGH].reshape(-1)
    src_pad = jnp.pad(src, (0, E_PAD - E_EDGES))

    posw0 = jnp.concatenate([pos16, hw0], axis=1)
    gath0 = _sc_gather(posw0, src_pad)
    e = _edge_call(gath0, pos16, Wq1, bq1.reshape(1, HID), Wq2,
                   bq2.reshape(1, HID))

    Wd2p = jnp.pad(Wd2, ((0, 0), (0, HID - 3)))
    bd2p = jnp.pad(bd2, (0, HID - 3)).reshape(1, HID)

    h, hw = h0, hw0
    for l in range(N_LAYERS):
        if l == 0:
            g, g_col = gath0, 1
        else:
            g, g_col = _sc_gather(hw, src_pad), 0
        last = l == N_LAYERS - 1
        if last:
            Wn, bn = Wd1, bd1.reshape(1, HID)
            Wn2, bn2 = Wd2p, bd2p
        else:
            Wn, bn = Wm1[l + 1, :HID, :], bm1[l + 1].reshape(1, HID)
            Wn2, bn2 = Wd2p, bd2p
        h, hw = _layer_call(
            g, g_col, e, h, Wm1[l, HID:, :], Wm2[l], bm2[l].reshape(1, HID),
            Wu[l], bu[l].reshape(1, HID), Wn, bn, Wn2, bn2, last)

    return hw[:, :3].astype(f32)
